# baseline (speedup 1.0000x reference)
import functools
import jax
import jax.numpy as jnp
from jax import lax
from jax.experimental import pallas as pl
from jax.experimental.pallas import tpu as pltpu
from jax.experimental.pallas import tpu_sc as plsc

N = 10000
E = 160000
HV = 128
HE = 64
PQ = 3
ME = 64
MM = 128
NM = 64
TAU = 0.25

BN = 1000
BE = 2000

NP = 10112
E2 = 163840
NW = 32
CHUNK = 128
NCH = E2 // (NW * CHUNK)
STRIPE = NP // 16


def _row_spec(block_rows, cols):
    return pl.BlockSpec((block_rows, cols), lambda i: (i, 0))


def _full_spec(shape):
    return pl.BlockSpec(shape, lambda *a: tuple(0 for _ in shape))


def _init_nodes_body(atom_ref, wiv_ref, biv_ref, wp_ref, wq_ref,
                     wmu_ref, wmv_ref, wd_ref, whu_ref, whv_ref,
                     hv_ref, p_ref, q_ref, tu_ref, tv_ref):
    hv = jnp.tanh(jnp.dot(atom_ref[...], wiv_ref[...],
                          preferred_element_type=jnp.float32) + biv_ref[...])
    hv_ref[...] = hv
    p = jnp.dot(hv, wp_ref[...], preferred_element_type=jnp.float32)
    q = jnp.dot(hv, wq_ref[...], preferred_element_type=jnp.float32)
    p_ref[...] = p
    q_ref[...] = q
    qd = jnp.dot(q, wd_ref[...], preferred_element_type=jnp.float32)
    a = jnp.dot(hv, wmu_ref[...], preferred_element_type=jnp.float32) + qd
    b = jnp.dot(hv, wmv_ref[...], preferred_element_type=jnp.float32) - qd
    cu = jnp.dot(hv, whu_ref[...], preferred_element_type=jnp.float32)
    cv = jnp.dot(hv, whv_ref[...], preferred_element_type=jnp.float32)
    tu_ref[...] = jnp.concatenate(
        [a, jnp.broadcast_to(cu, (a.shape[0], 16))], axis=1)
    tv_ref[...] = jnp.concatenate(
        [b, jnp.broadcast_to(cv, (b.shape[0], 16))], axis=1)


def _init_nodes(atom_ftr, wiv, biv, wp8, wq8, wmu, wmv, wd8, whu, whv):
    return pl.pallas_call(
        _init_nodes_body,
        grid=(N // BN,),
        in_specs=[
            _row_spec(BN, HV), _full_spec((HV, HV)), _full_spec((1, HV)),
            _full_spec((HV, 16)), _full_spec((HV, 16)),
            _full_spec((HV, ME)), _full_spec((HV, ME)), _full_spec((16, ME)),
            _full_spec((HV, 1)), _full_spec((HV, 1)),
        ],
        out_specs=[
            _row_spec(BN, HV), _row_spec(BN, 16), _row_spec(BN, 16),
            _row_spec(BN, 80), _row_spec(BN, 80),
        ],
        out_shape=[
            jax.ShapeDtypeStruct((N, HV), jnp.float32),
            jax.ShapeDtypeStruct((N, 16), jnp.float32),
            jax.ShapeDtypeStruct((N, 16), jnp.float32),
            jax.ShapeDtypeStruct((N, 80), jnp.float32),
            jax.ShapeDtypeStruct((N, 80), jnp.float32),
        ],
    )(atom_ftr, wiv, biv, wp8, wq8, wmu, wmv, wd8, whu, whv)


def _node_tables_body(hv_ref, q_ref, wmu_ref, wmv_ref, wd_ref, whu_ref,
                      whv_ref, tu_ref, tv_ref):
    hv = hv_ref[...]
    qd = jnp.dot(q_ref[...], wd_ref[...], preferred_element_type=jnp.float32)
    a = jnp.dot(hv, wmu_ref[...], preferred_element_type=jnp.float32) + qd
    b = jnp.dot(hv, wmv_ref[...], preferred_element_type=jnp.float32) - qd
    cu = jnp.dot(hv, whu_ref[...], preferred_element_type=jnp.float32)
    cv = jnp.dot(hv, whv_ref[...], preferred_element_type=jnp.float32)
    tu_ref[...] = jnp.concatenate(
        [a, jnp.broadcast_to(cu, (a.shape[0], 16))], axis=1)
    tv_ref[...] = jnp.concatenate(
        [b, jnp.broadcast_to(cv, (b.shape[0], 16))], axis=1)


def _node_tables(hv, q8, wmu, wmv, wd8, whu, whv):
    return pl.pallas_call(
        _node_tables_body,
        grid=(N // BN,),
        in_specs=[
            _row_spec(BN, HV), _row_spec(BN, 16),
            _full_spec((HV, ME)), _full_spec((HV, ME)), _full_spec((16, ME)),
            _full_spec((HV, 1)), _full_spec((HV, 1)),
        ],
        out_specs=[_row_spec(BN, 80), _row_spec(BN, 80)],
        out_shape=[
            jax.ShapeDtypeStruct((N, 80), jnp.float32),
            jax.ShapeDtypeStruct((N, 80), jnp.float32),
        ],
    )(hv, q8, wmu, wmv, wd8, whu, whv)


def _init_edges_body(bond_ref, wie_ref, bie_ref, wme_ref, whe_ref,
                     he_ref, mhe_ref, ce_ref):
    he = jnp.tanh(jnp.dot(bond_ref[...], wie_ref[...],
                          preferred_element_type=jnp.float32) + bie_ref[...])
    he_ref[...] = he
    mhe_ref[...] = jnp.dot(he, wme_ref[...], preferred_element_type=jnp.float32)
    ce = jnp.dot(he, whe_ref[...], preferred_element_type=jnp.float32)
    ce_ref[...] = jnp.broadcast_to(ce, (ce.shape[0], 16))


def _init_edges(bond_ftr, wie, bie, wme, whe):
    return pl.pallas_call(
        _init_edges_body,
        grid=(E // BE,),
        in_specs=[
            _row_spec(BE, 16), _full_spec((16, HE)), _full_spec((1, HE)),
            _full_spec((HE, ME)), _full_spec((HE, 1)),
        ],
        out_specs=[_row_spec(BE, HE), _row_spec(BE, ME), _row_spec(BE, 16)],
        out_shape=[
            jax.ShapeDtypeStruct((E, HE), jnp.float32),
            jax.ShapeDtypeStruct((E, ME), jnp.float32),
            jax.ShapeDtypeStruct((E, 16), jnp.float32),
        ],
    )(bond_ftr, wie, bie, wme, whe)


def _upd_v_body(hv_ref, agga_ref, aggb_ref, wv1_ref, wv2a_ref, wv2b_ref,
                out_ref):
    out_ref[...] = jax.nn.relu(
        jnp.dot(hv_ref[...], wv1_ref[...], preferred_element_type=jnp.float32)
        + jnp.dot(agga_ref[...], wv2a_ref[...],
                  preferred_element_type=jnp.float32)
        + jnp.dot(aggb_ref[...], wv2b_ref[...],
                  preferred_element_type=jnp.float32))


def _upd_v(hv, agga, aggb, wv1, wv2a, wv2b):
    return pl.pallas_call(
        _upd_v_body,
        grid=(N // BN,),
        in_specs=[
            _row_spec(BN, HV), _row_spec(BN, HME), _row_spec(BN, HME),
            _full_spec((HV, HV)), _full_spec((HME, HV)),
            _full_spec((HME, HV)),
        ],
        out_specs=_row_spec(BN, HV),
        out_shape=jax.ShapeDtypeStruct((N, HV), jnp.float32),
    )(hv, agga, aggb, wv1, wv2a, wv2b)


def _upd_e_proj_body(he_ref, ma_ref, mb_ref, wa_ref, wb1_ref, wb2_ref,
                     wc_ref, wd_ref, the_ref, mhe_ref, ce_ref):
    t_he = jax.nn.relu(
        jnp.dot(he_ref[...], wa_ref[...], preferred_element_type=jnp.float32)
        + jnp.dot(ma_ref[...], wb1_ref[...],
                  preferred_element_type=jnp.float32)
        + jnp.dot(mb_ref[...], wb2_ref[...],
                  preferred_element_type=jnp.float32))
    the_ref[...] = t_he
    mhe_ref[...] = jnp.dot(t_he, wc_ref[...], preferred_element_type=jnp.float32)
    ce = jnp.dot(t_he, wd_ref[...], preferred_element_type=jnp.float32)
    ce_ref[...] = jnp.broadcast_to(ce, (ce.shape[0], 16))


def _upd_e_proj(he, ma, mb, wa, wb1, wb2, wc, wd):
    return pl.pallas_call(
        _upd_e_proj_body,
        grid=(E // BE,),
        in_specs=[
            _row_spec(BE, HE), _row_spec(BE, HME), _row_spec(BE, HME),
            _full_spec((HE, HE)), _full_spec((HME, HE)),
            _full_spec((HME, HE)),
            _full_spec((HE, ME)), _full_spec((HE, 1)),
        ],
        out_specs=[_row_spec(BE, HE), _row_spec(BE, ME), _row_spec(BE, 16)],
        out_shape=[
            jax.ShapeDtypeStruct((E, HE), jnp.float32),
            jax.ShapeDtypeStruct((E, ME), jnp.float32),
            jax.ShapeDtypeStruct((E, 16), jnp.float32),
        ],
    )(he, ma, mb, wa, wb1, wb2, wc, wd)


def _readout_body(hv_ref, p_ref, q_ref, mid_ref, wfp_ref, wahv_ref, wap_ref,
                  waq_ref, wc1_ref, bc1_ref, wc2_ref, bc2_ref,
                  fp_ref, conf_ref):
    hv = hv_ref[...]
    hvp = jnp.dot(hv, wfp_ref[...], preferred_element_type=jnp.float32)
    a = jax.nn.sigmoid(
        jnp.dot(hv, wahv_ref[...], preferred_element_type=jnp.float32)
        + jnp.dot(p_ref[...], wap_ref[...], preferred_element_type=jnp.float32)
        + jnp.dot(q_ref[...], waq_ref[...], preferred_element_type=jnp.float32))
    mids = mid_ref[...]
    oh = (mids == lax.broadcasted_iota(jnp.int32, (N, NM), 1)).astype(jnp.float32)
    hm = lax.dot_general(oh, a * hvp, (((0,), (0,)), ((), ())),
                         preferred_element_type=jnp.float32)
    for _ in range(2):
        g = jnp.dot(oh, hm, preferred_element_type=jnp.float32)
        a2 = jax.nn.sigmoid(jnp.sum(hvp * g, axis=1, keepdims=True))
        hm = lax.dot_general(oh, a2 * hvp, (((0,), (0,)), ((), ())),
                             preferred_element_type=jnp.float32)
    fp_ref[...] = hm
    conf_ref[...] = (
        jnp.dot(jax.nn.relu(
            jnp.dot(q_ref[...], wc1_ref[...], preferred_element_type=jnp.float32)
            + bc1_ref[...]), wc2_ref[...], preferred_element_type=jnp.float32)
        + bc2_ref[...])


def _readout(hv, p8, q8, mid, wfp, wahv, wap8, waq8, wc18, bc1, wc2, bc38):
    return pl.pallas_call(
        _readout_body,
        in_specs=[
            _full_spec((N, HV)), _full_spec((N, 16)), _full_spec((N, 16)),
            _full_spec((N, 1)), _full_spec((HV, MM)), _full_spec((HV, 1)),
            _full_spec((16, 1)), _full_spec((16, 1)),
            _full_spec((16, MM)), _full_spec((1, MM)),
            _full_spec((MM, 8)), _full_spec((1, 8)),
        ],
        out_specs=[_full_spec((NM, MM)), _full_spec((N, 8))],
        out_shape=[
            jax.ShapeDtypeStruct((NM, MM), jnp.float32),
            jax.ShapeDtypeStruct((N, 8), jnp.float32),
        ],
    )(hv, p8, q8, mid, wfp, wahv, wap8, waq8, wc18, bc1, wc2, bc38)


_MESH = plsc.VectorSubcoreMesh(core_axis_name="c", subcore_axis_name="s")
_MESH1 = plsc.VectorSubcoreMesh(core_axis_name="c", subcore_axis_name="s",
                                num_cores=1)


def _copy_idx_row(src2d, j, dst1d):
    for k in range(CHUNK // 16):
        sl = pl.ds(k * 16, 16)
        dst1d[sl] = src2d[j, sl]


HME = ME // 2


@functools.lru_cache(maxsize=None)
def _sc_edge_msg_fn():
    out_type = [
        jax.ShapeDtypeStruct((E2, HME), jnp.float32),
        jax.ShapeDtypeStruct((E2, HME), jnp.float32),
        jax.ShapeDtypeStruct((E2, 16), jnp.float32),
        jax.ShapeDtypeStruct((NP, HME), jnp.float32),
        jax.ShapeDtypeStruct((NP, HME), jnp.float32),
    ]

    def body(tu_h, tv_h, mhe_h, ce_h, u2_h, v2_h, z_h,
             ma_out, mb_out, coef_out, agga_out, aggb_out,
             uix, vix, u1, v1, au, bv, mhv, cev, mva, mvb, cfv, shared,
             sg1, sg2, sl_):
        sid = lax.axis_index("s")
        wid = sid
        stripe = pl.ds(sid * STRIPE, STRIPE)
        pltpu.sync_copy(z_h.at[stripe], shared.at[stripe])
        pltpu.sync_copy(u2_h.at[pl.ds(wid * NCH_H, NCH_H)], uix)
        pltpu.sync_copy(v2_h.at[pl.ds(wid * NCH_H, NCH_H)], vix)
        plsc.subcore_barrier()

        def fire(b, j):
            base = wid * (NCH_H * CHUNK) + j * CHUNK
            _copy_idx_row(uix, j, u1[b])
            _copy_idx_row(vix, j, v1[b])
            pltpu.async_copy(tu_h.at[u1[b]], au[b], sg1[b])
            pltpu.async_copy(tv_h.at[v1[b]], bv[b], sg2[b])
            pltpu.async_copy(mhe_h.at[pl.ds(base, CHUNK)], mhv[b], sl_[b])
            pltpu.async_copy(ce_h.at[pl.ds(base, CHUNK)], cev[b], sl_[b])

        def process(b, j, do_fire):
            pltpu.make_async_copy(tu_h.at[pl.ds(0, CHUNK)], au[b],
                                  sg1[b]).wait()
            pltpu.make_async_copy(tv_h.at[pl.ds(0, CHUNK)], bv[b],
                                  sg2[b]).wait()
            pltpu.make_async_copy(mhe_h.at[pl.ds(0, CHUNK)], mhv[b],
                                  sl_[b]).wait()
            pltpu.make_async_copy(ce_h.at[pl.ds(0, CHUNK)], cev[b],
                                  sl_[b]).wait()

            def row(r2, c2):
                for rr in range(2):
                    r = r2 * 2 + rr
                    for cc in range(2):
                        sl = pl.ds(cc * 16, 16)
                        mva[b][r, sl] = jnp.maximum(
                            au[b][r, sl] + bv[b][r, sl] + mhv[b][r, sl], 0.0)
                    for cc in range(2):
                        sl = pl.ds(HME + cc * 16, 16)
                        so = pl.ds(cc * 16, 16)
                        mvb[b][r, so] = jnp.maximum(
                            au[b][r, sl] + bv[b][r, sl] + mhv[b][r, sl], 0.0)
                    tl = pl.ds(ME, 16)
                    s = au[b][r, tl] + bv[b][r, tl] + cev[b][r, pl.ds(0, 16)]
                    cfv[b][r, pl.ds(0, 16)] = (
                        1.0 - 2.0 / (jnp.exp(2.0 * s) + 1.0))
                return c2

            lax.fori_loop(0, CHUNK // 2, row, 0)
            pltpu.sync_copy(mva[b], shared.at[v1[b]], add=True)
            if do_fire:
                fire(b, j + 2)
            base = wid * (NCH_H * CHUNK) + j * CHUNK
            pltpu.sync_copy(mva[b], ma_out.at[pl.ds(base, CHUNK)])
            pltpu.sync_copy(mvb[b], mb_out.at[pl.ds(base, CHUNK)])
            pltpu.sync_copy(cfv[b], coef_out.at[pl.ds(base, CHUNK)])

        fire(0, 0)
        fire(1, 1)

        def pair(jj, carry):
            process(0, 2 * jj, True)
            process(1, 2 * jj + 1, True)
            return carry

        lax.fori_loop(0, (NCH_H - 2) // 2, pair, 0)
        process(0, NCH_H - 2, False)
        process(1, NCH_H - 1, False)
        plsc.subcore_barrier()
        pltpu.sync_copy(shared.at[stripe], agga_out.at[stripe])
        pltpu.sync_copy(z_h.at[stripe], shared.at[stripe])
        plsc.subcore_barrier()

        def fire_b(b, j):
            base = wid * (NCH_H * CHUNK) + j * CHUNK
            _copy_idx_row(vix, j, v1[b])
            pltpu.async_copy(mb_out.at[pl.ds(base, CHUNK)], mvb[b], sl_[b])

        def process_b(b, j, do_fire):
            pltpu.make_async_copy(mb_out.at[pl.ds(0, CHUNK)], mvb[b],
                                  sl_[b]).wait()
            pltpu.sync_copy(mvb[b], shared.at[v1[b]], add=True)
            if do_fire:
                fire_b(b, j + 2)

        fire_b(0, 0)
        fire_b(1, 1)

        def pair_b(jj, carry):
            process_b(0, 2 * jj, True)
            process_b(1, 2 * jj + 1, True)
            return carry

        lax.fori_loop(0, (NCH_H - 2) // 2, pair_b, 0)
        process_b(0, NCH_H - 2, False)
        process_b(1, NCH_H - 1, False)
        plsc.subcore_barrier()
        pltpu.sync_copy(shared.at[stripe], aggb_out.at[stripe])

    scratch = [
        pltpu.VMEM((NCH_H, CHUNK), jnp.int32),
        pltpu.VMEM((NCH_H, CHUNK), jnp.int32),
        [pltpu.VMEM((CHUNK,), jnp.int32)] * 2,
        [pltpu.VMEM((CHUNK,), jnp.int32)] * 2,
        [pltpu.VMEM((CHUNK, 80), jnp.float32)] * 2,
        [pltpu.VMEM((CHUNK, 80), jnp.float32)] * 2,
        [pltpu.VMEM((CHUNK, ME), jnp.float32)] * 2,
        [pltpu.VMEM((CHUNK, 16), jnp.float32)] * 2,
        [pltpu.VMEM((CHUNK, HME), jnp.float32)] * 2,
        [pltpu.VMEM((CHUNK, HME), jnp.float32)] * 2,
        [pltpu.VMEM((CHUNK, 16), jnp.float32)] * 2,
        pltpu.VMEM_SHARED((NP, HME), jnp.float32),
        [pltpu.SemaphoreType.DMA] * 2,
        [pltpu.SemaphoreType.DMA] * 2,
        [pltpu.SemaphoreType.DMA] * 2,
    ]
    return pl.kernel(body, out_type=out_type, mesh=_MESH1,
                     scratch_types=scratch,
                     compiler_params=pltpu.CompilerParams(
                         use_tc_tiling_on_sc=False))


def _sc_edge_msg(tu, tv, mhe, ce, u2d, v2d, zeros32):
    return _sc_edge_msg_fn()(tu, tv, mhe, ce, u2d, v2d, zeros32)


NCH_H = E2 // (16 * CHUNK)


@functools.lru_cache(maxsize=None)
def _sc_ham_layer_fn():
    out_type = [
        jax.ShapeDtypeStruct((NP, 16), jnp.float32),
        jax.ShapeDtypeStruct((NP, 16), jnp.float32),
    ]

    def body(q_h, p_h, tm_h, cf_h, u2_h, v2_h, z_h, qf_h, pf_h,
             uix, vix, u1, v1, qu, qv, wv, cfv, qs, ps, tms, fs, shared,
             sg1, sg2, sl_):
        cid = lax.axis_index("c")
        sid = lax.axis_index("s")

        @pl.when(cid == 0)
        def _core0():
            stripe = pl.ds(sid * STRIPE, STRIPE)
            pltpu.sync_copy(z_h.at[stripe], shared.at[stripe])
            pltpu.sync_copy(u2_h.at[pl.ds(sid * NCH_H, NCH_H)], uix)
            pltpu.sync_copy(v2_h.at[pl.ds(sid * NCH_H, NCH_H)], vix)
            pltpu.sync_copy(q_h.at[stripe], qs)
            pltpu.sync_copy(p_h.at[stripe], ps)
            pltpu.sync_copy(tm_h.at[stripe], tms)
            plsc.subcore_barrier()

            for it in range(4):
                qsrc = q_h if it == 0 else qf_h

                def fire(b, j, qsrc=qsrc):
                    base = sid * (NCH_H * CHUNK) + j * CHUNK
                    _copy_idx_row(uix, j, u1[b])
                    _copy_idx_row(vix, j, v1[b])
                    pltpu.async_copy(qsrc.at[u1[b]], qu[b], sg1[b])
                    pltpu.async_copy(qsrc.at[v1[b]], qv[b], sg2[b])
                    pltpu.async_copy(cf_h.at[pl.ds(base, CHUNK)], cfv[b],
                                     sl_[b])

                def process(b, j, do_fire, qsrc=qsrc):
                    pltpu.make_async_copy(qsrc.at[pl.ds(0, CHUNK)], qu[b],
                                          sg1[b]).wait()
                    pltpu.make_async_copy(qsrc.at[pl.ds(0, CHUNK)], qv[b],
                                          sg2[b]).wait()
                    pltpu.make_async_copy(cf_h.at[pl.ds(0, CHUNK)], cfv[b],
                                          sl_[b]).wait()

                    def row(r4, c2):
                        for rr in range(4):
                            r = r4 * 4 + rr
                            sl = pl.ds(0, 16)
                            wv[b][r, sl] = cfv[b][r, sl] * (
                                qv[b][r, sl] - qu[b][r, sl])
                        return c2

                    lax.fori_loop(0, CHUNK // 4, row, 0)
                    pltpu.sync_copy(wv[b], shared.at[u1[b]], add=True)
                    if do_fire:
                        fire(b, j + 2)

                fire(0, 0)
                fire(1, 1)

                def pair(jj, carry):
                    process(0, 2 * jj, True)
                    process(1, 2 * jj + 1, True)
                    return carry

                lax.fori_loop(0, (NCH_H - 2) // 2, pair, 0)
                process(0, NCH_H - 2, False)
                process(1, NCH_H - 1, False)
                plsc.subcore_barrier()

                pltpu.sync_copy(shared.at[stripe], fs)

                def urow(r4, c2):
                    for rr in range(4):
                        r = r4 * 4 + rr
                        sl = pl.ds(0, 16)
                        pold = ps[r, sl]
                        qs[r, sl] = qs[r, sl] + pold * tms[r, sl]
                        ps[r, sl] = ((1.0 - 0.1 * TAU) * pold
                                     + TAU * fs[r, sl])
                    return c2

                lax.fori_loop(0, STRIPE // 4, urow, 0)
                pltpu.sync_copy(qs, qf_h.at[stripe])
                if it < 3:
                    pltpu.sync_copy(z_h.at[stripe], shared.at[stripe])
                plsc.subcore_barrier()

            pltpu.sync_copy(ps, pf_h.at[stripe])

    scratch = [
        pltpu.VMEM((NCH_H, CHUNK), jnp.int32),
        pltpu.VMEM((NCH_H, CHUNK), jnp.int32),
        [pltpu.VMEM((CHUNK,), jnp.int32)] * 2,
        [pltpu.VMEM((CHUNK,), jnp.int32)] * 2,
        [pltpu.VMEM((CHUNK, 16), jnp.float32)] * 2,
        [pltpu.VMEM((CHUNK, 16), jnp.float32)] * 2,
        [pltpu.VMEM((CHUNK, 16), jnp.float32)] * 2,
        [pltpu.VMEM((CHUNK, 16), jnp.float32)] * 2,
        pltpu.VMEM((STRIPE, 16), jnp.float32),
        pltpu.VMEM((STRIPE, 16), jnp.float32),
        pltpu.VMEM((STRIPE, 16), jnp.float32),
        pltpu.VMEM((STRIPE, 16), jnp.float32),
        pltpu.VMEM_SHARED((NP, 16), jnp.float32),
        [pltpu.SemaphoreType.DMA] * 2,
        [pltpu.SemaphoreType.DMA] * 2,
        [pltpu.SemaphoreType.DMA] * 2,
    ]
    return pl.kernel(body, out_type=out_type, mesh=_MESH1,
                     scratch_types=scratch,
                     compiler_params=pltpu.CompilerParams(
                         use_tc_tiling_on_sc=False))


def _sc_ham_layer(q16, p16, tm16, coef, u2d, v2d, zeros16):
    return _sc_ham_layer_fn()(q16, p16, tm16, coef, u2d, v2d, zeros16)


def kernel(atom_ftr, bond_ftr, massive, edge_index, mol_ids,
           W_init_v, b_init_v, W_init_e, b_init_e, W_p, W_q, W_msg,
           W_upd_v, W_upd_e, W_ham, W_att, W_fp, W_c1, b_c1, W_c2, b_c2):
    def pad_cols(w, cols):
        return jnp.pad(w, ((0, 0), (0, cols - w.shape[1])))

    def pad_rows(w, rows):
        return jnp.pad(w, ((0, rows - w.shape[0]), (0, 0)))

    def pad_nodes(x):
        return jnp.pad(x, ((0, NP - N), (0, 0)))

    def pad_edges(x):
        return jnp.pad(x, ((0, E2 - E), (0, 0)))

    u2d = jnp.concatenate(
        [edge_index[0].astype(jnp.int32),
         jnp.full((E2 - E,), N, jnp.int32)]).reshape(E2 // CHUNK, CHUNK)
    v2d = jnp.concatenate(
        [edge_index[1].astype(jnp.int32),
         jnp.full((E2 - E,), N, jnp.int32)]).reshape(E2 // CHUNK, CHUNK)
    zeros32 = jnp.zeros((NP, HME), jnp.float32)
    zeros16 = jnp.zeros((NP, 16), jnp.float32)
    mass_pad = jnp.pad(massive, ((0, NP - N), (0, 0)), constant_values=1.0)
    tm16 = jnp.broadcast_to(TAU / mass_pad, (NP, 16))

    wp16 = pad_cols(W_p, 16)
    wq16 = pad_cols(W_q, 16)
    wd16 = [pad_rows(W_msg[i][2 * HV + HE:], 16) for i in range(2)]
    wmu = [W_msg[i][:HV] for i in range(2)]
    wmv = [W_msg[i][HV:2 * HV] for i in range(2)]
    whu = [W_ham[i][:HV] for i in range(2)]
    whv = [W_ham[i][HV:2 * HV] for i in range(2)]

    hv, p16, q16, tu, tv = _init_nodes(
        atom_ftr, W_init_v, b_init_v[None], wp16, wq16,
        wmu[0], wmv[0], wd16[0], whu[0], whv[0])
    p16 = pad_nodes(p16)
    q16 = pad_nodes(q16)
    he, mhe, ce = _init_edges(
        bond_ftr, W_init_e, b_init_e[None],
        W_msg[0][2 * HV:2 * HV + HE], W_ham[0][2 * HV:])

    nxt = [1, 1]
    xs = dict(
        wuv1=jnp.stack([W_upd_v[i][:HV] for i in range(2)]),
        wuv2=jnp.stack([W_upd_v[i][HV:] for i in range(2)]),
        wa=jnp.stack([W_upd_e[i][:HE] for i in range(2)]),
        wb=jnp.stack([W_upd_e[i][HE:] for i in range(2)]),
        wc=jnp.stack([W_msg[j][2 * HV:2 * HV + HE] for j in nxt]),
        wdh=jnp.stack([W_ham[j][2 * HV:] for j in nxt]),
        wmu_n=jnp.stack([wmu[j] for j in nxt]),
        wmv_n=jnp.stack([wmv[j] for j in nxt]),
        wd16_n=jnp.stack([wd16[j] for j in nxt]),
        whu_n=jnp.stack([whu[j] for j in nxt]),
        whv_n=jnp.stack([whv[j] for j in nxt]),
    )

    def layer_body(carry, ws):
        hv, q16, p16, tu, tv, he, mhe, ce = carry
        ma, mb, coef, agga, aggb = _sc_edge_msg(
            pad_nodes(tu), pad_nodes(tv), pad_edges(mhe), pad_edges(ce),
            u2d, v2d, zeros32)
        t_hv = _upd_v(hv, agga[:N], aggb[:N], ws['wuv1'],
                      ws['wuv2'][:HME], ws['wuv2'][HME:])
        t_he, mhe_n, ce_n = _upd_e_proj(he, ma[:E], mb[:E], ws['wa'],
                                        ws['wb'][:HME], ws['wb'][HME:],
                                        ws['wc'], ws['wdh'])
        q16, p16 = _sc_ham_layer(q16, p16, tm16, coef, u2d, v2d, zeros16)
        tu_n, tv_n = _node_tables(t_hv, q16[:N], ws['wmu_n'], ws['wmv_n'],
                                  ws['wd16_n'], ws['whu_n'], ws['whv_n'])
        return (t_hv, q16, p16, tu_n, tv_n, t_he, mhe_n, ce_n), None

    carry, _ = lax.scan(layer_body, (hv, q16, p16, tu, tv, he, mhe, ce), xs,
                        length=2)
    hv, q16, p16 = carry[0], carry[1], carry[2]

    fp, conf8 = _readout(
        hv, p16[:N], q16[:N], mol_ids[:, None].astype(jnp.int32),
        W_fp, W_att[:HV], pad_rows(W_att[HV:HV + PQ], 16),
        pad_rows(W_att[HV + PQ:], 16), pad_rows(W_c1, 16), b_c1[None],
        pad_cols(W_c2, 8), pad_cols(b_c2[None], 8))
    return (fp, conf8[:, :PQ])


# speedup vs baseline: 1.1059x; 1.1059x over previous
import functools
import jax
import jax.numpy as jnp
from jax import lax
from jax.experimental import pallas as pl
from jax.experimental.pallas import tpu as pltpu
from jax.experimental.pallas import tpu_sc as plsc

N = 10000
E = 160000
HV = 128
HE = 64
PQ = 3
ME = 64
MM = 128
NM = 64
TAU = 0.25

BN = 1000
BE = 2000

NP = 10112
E2 = 163840
NW = 32
CHUNK = 128
NCH = E2 // (NW * CHUNK)
STRIPE = NP // 16


def _row_spec(block_rows, cols):
    return pl.BlockSpec((block_rows, cols), lambda i: (i, 0))


def _full_spec(shape):
    return pl.BlockSpec(shape, lambda *a: tuple(0 for _ in shape))


def _init_nodes_body(atom_ref, wiv_ref, biv_ref, wp_ref, wq_ref,
                     wmu_ref, wmv_ref, wd_ref, whu_ref, whv_ref,
                     hv_ref, p_ref, q_ref, tu_ref, tv_ref):
    hv = jnp.tanh(jnp.dot(atom_ref[...], wiv_ref[...],
                          preferred_element_type=jnp.float32) + biv_ref[...])
    hv_ref[...] = hv
    p = jnp.dot(hv, wp_ref[...], preferred_element_type=jnp.float32)
    q = jnp.dot(hv, wq_ref[...], preferred_element_type=jnp.float32)
    p_ref[...] = p
    q_ref[...] = q
    qd = jnp.dot(q, wd_ref[...], preferred_element_type=jnp.float32)
    a = jnp.dot(hv, wmu_ref[...], preferred_element_type=jnp.float32) + qd
    b = jnp.dot(hv, wmv_ref[...], preferred_element_type=jnp.float32) - qd
    cu = jnp.dot(hv, whu_ref[...], preferred_element_type=jnp.float32)
    cv = jnp.dot(hv, whv_ref[...], preferred_element_type=jnp.float32)
    tu_ref[...] = jnp.concatenate(
        [a, jnp.broadcast_to(cu, (a.shape[0], 16))], axis=1)
    tv_ref[...] = jnp.concatenate(
        [b, jnp.broadcast_to(cv, (b.shape[0], 16))], axis=1)


def _init_nodes(atom_ftr, wiv, biv, wp8, wq8, wmu, wmv, wd8, whu, whv):
    return pl.pallas_call(
        _init_nodes_body,
        grid=(N // BN,),
        in_specs=[
            _row_spec(BN, HV), _full_spec((HV, HV)), _full_spec((1, HV)),
            _full_spec((HV, 16)), _full_spec((HV, 16)),
            _full_spec((HV, ME)), _full_spec((HV, ME)), _full_spec((16, ME)),
            _full_spec((HV, 1)), _full_spec((HV, 1)),
        ],
        out_specs=[
            _row_spec(BN, HV), _row_spec(BN, 16), _row_spec(BN, 16),
            _row_spec(BN, 80), _row_spec(BN, 80),
        ],
        out_shape=[
            jax.ShapeDtypeStruct((N, HV), jnp.float32),
            jax.ShapeDtypeStruct((N, 16), jnp.float32),
            jax.ShapeDtypeStruct((N, 16), jnp.float32),
            jax.ShapeDtypeStruct((N, 80), jnp.float32),
            jax.ShapeDtypeStruct((N, 80), jnp.float32),
        ],
    )(atom_ftr, wiv, biv, wp8, wq8, wmu, wmv, wd8, whu, whv)


def _node_tables_body(hv_ref, q_ref, wmu_ref, wmv_ref, wd_ref, whu_ref,
                      whv_ref, tu_ref, tv_ref):
    hv = hv_ref[...]
    qd = jnp.dot(q_ref[...], wd_ref[...], preferred_element_type=jnp.float32)
    a = jnp.dot(hv, wmu_ref[...], preferred_element_type=jnp.float32) + qd
    b = jnp.dot(hv, wmv_ref[...], preferred_element_type=jnp.float32) - qd
    cu = jnp.dot(hv, whu_ref[...], preferred_element_type=jnp.float32)
    cv = jnp.dot(hv, whv_ref[...], preferred_element_type=jnp.float32)
    tu_ref[...] = jnp.concatenate(
        [a, jnp.broadcast_to(cu, (a.shape[0], 16))], axis=1)
    tv_ref[...] = jnp.concatenate(
        [b, jnp.broadcast_to(cv, (b.shape[0], 16))], axis=1)


def _node_tables(hv, q8, wmu, wmv, wd8, whu, whv):
    return pl.pallas_call(
        _node_tables_body,
        grid=(N // BN,),
        in_specs=[
            _row_spec(BN, HV), _row_spec(BN, 16),
            _full_spec((HV, ME)), _full_spec((HV, ME)), _full_spec((16, ME)),
            _full_spec((HV, 1)), _full_spec((HV, 1)),
        ],
        out_specs=[_row_spec(BN, 80), _row_spec(BN, 80)],
        out_shape=[
            jax.ShapeDtypeStruct((N, 80), jnp.float32),
            jax.ShapeDtypeStruct((N, 80), jnp.float32),
        ],
    )(hv, q8, wmu, wmv, wd8, whu, whv)


def _init_edges_body(bond_ref, wie_ref, bie_ref, wme_ref, whe_ref,
                     he_ref, mhe_ref, ce_ref):
    he = jnp.tanh(jnp.dot(bond_ref[...], wie_ref[...],
                          preferred_element_type=jnp.float32) + bie_ref[...])
    he_ref[...] = he
    mhe_ref[...] = jnp.dot(he, wme_ref[...], preferred_element_type=jnp.float32)
    ce = jnp.dot(he, whe_ref[...], preferred_element_type=jnp.float32)
    ce_ref[...] = jnp.broadcast_to(ce, (ce.shape[0], 16))


def _init_edges(bond_ftr, wie, bie, wme, whe):
    return pl.pallas_call(
        _init_edges_body,
        grid=(E // BE,),
        in_specs=[
            _row_spec(BE, 16), _full_spec((16, HE)), _full_spec((1, HE)),
            _full_spec((HE, ME)), _full_spec((HE, 1)),
        ],
        out_specs=[_row_spec(BE, HE), _row_spec(BE, ME), _row_spec(BE, 16)],
        out_shape=[
            jax.ShapeDtypeStruct((E, HE), jnp.float32),
            jax.ShapeDtypeStruct((E, ME), jnp.float32),
            jax.ShapeDtypeStruct((E, 16), jnp.float32),
        ],
    )(bond_ftr, wie, bie, wme, whe)


def _upd_v_body(hv_ref, agga_ref, aggb_ref, wv1_ref, wv2a_ref, wv2b_ref,
                out_ref):
    out_ref[...] = jax.nn.relu(
        jnp.dot(hv_ref[...], wv1_ref[...], preferred_element_type=jnp.float32)
        + jnp.dot(agga_ref[0] + agga_ref[1], wv2a_ref[...],
                  preferred_element_type=jnp.float32)
        + jnp.dot(aggb_ref[0] + aggb_ref[1], wv2b_ref[...],
                  preferred_element_type=jnp.float32))


def _upd_v(hv, agga, aggb, wv1, wv2a, wv2b):
    return pl.pallas_call(
        _upd_v_body,
        grid=(N // BN,),
        in_specs=[
            _row_spec(BN, HV),
            pl.BlockSpec((2, BN, HME), lambda i: (0, i, 0)),
            pl.BlockSpec((2, BN, HME), lambda i: (0, i, 0)),
            _full_spec((HV, HV)), _full_spec((HME, HV)),
            _full_spec((HME, HV)),
        ],
        out_specs=_row_spec(BN, HV),
        out_shape=jax.ShapeDtypeStruct((N, HV), jnp.float32),
    )(hv, agga, aggb, wv1, wv2a, wv2b)


def _upd_e_proj_body(he_ref, ma_ref, mb_ref, wa_ref, wb1_ref, wb2_ref,
                     wc_ref, wd_ref, the_ref, mhe_ref, ce_ref):
    t_he = jax.nn.relu(
        jnp.dot(he_ref[...], wa_ref[...], preferred_element_type=jnp.float32)
        + jnp.dot(ma_ref[...], wb1_ref[...],
                  preferred_element_type=jnp.float32)
        + jnp.dot(mb_ref[...], wb2_ref[...],
                  preferred_element_type=jnp.float32))
    the_ref[...] = t_he
    mhe_ref[...] = jnp.dot(t_he, wc_ref[...], preferred_element_type=jnp.float32)
    ce = jnp.dot(t_he, wd_ref[...], preferred_element_type=jnp.float32)
    ce_ref[...] = jnp.broadcast_to(ce, (ce.shape[0], 16))


def _upd_e_proj(he, ma, mb, wa, wb1, wb2, wc, wd):
    return pl.pallas_call(
        _upd_e_proj_body,
        grid=(E // BE,),
        in_specs=[
            _row_spec(BE, HE), _row_spec(BE, HME), _row_spec(BE, HME),
            _full_spec((HE, HE)), _full_spec((HME, HE)),
            _full_spec((HME, HE)),
            _full_spec((HE, ME)), _full_spec((HE, 1)),
        ],
        out_specs=[_row_spec(BE, HE), _row_spec(BE, ME), _row_spec(BE, 16)],
        out_shape=[
            jax.ShapeDtypeStruct((E, HE), jnp.float32),
            jax.ShapeDtypeStruct((E, ME), jnp.float32),
            jax.ShapeDtypeStruct((E, 16), jnp.float32),
        ],
    )(he, ma, mb, wa, wb1, wb2, wc, wd)


def _readout_body(hv_ref, p_ref, q_ref, mid_ref, wfp_ref, wahv_ref, wap_ref,
                  waq_ref, wc1_ref, bc1_ref, wc2_ref, bc2_ref,
                  fp_ref, conf_ref):
    hv = hv_ref[...]
    hvp = jnp.dot(hv, wfp_ref[...], preferred_element_type=jnp.float32)
    a = jax.nn.sigmoid(
        jnp.dot(hv, wahv_ref[...], preferred_element_type=jnp.float32)
        + jnp.dot(p_ref[...], wap_ref[...], preferred_element_type=jnp.float32)
        + jnp.dot(q_ref[...], waq_ref[...], preferred_element_type=jnp.float32))
    mids = mid_ref[...]
    oh = (mids == lax.broadcasted_iota(jnp.int32, (N, NM), 1)).astype(jnp.float32)
    hm = lax.dot_general(oh, a * hvp, (((0,), (0,)), ((), ())),
                         preferred_element_type=jnp.float32)
    for _ in range(2):
        g = jnp.dot(oh, hm, preferred_element_type=jnp.float32)
        a2 = jax.nn.sigmoid(jnp.sum(hvp * g, axis=1, keepdims=True))
        hm = lax.dot_general(oh, a2 * hvp, (((0,), (0,)), ((), ())),
                             preferred_element_type=jnp.float32)
    fp_ref[...] = hm
    conf_ref[...] = (
        jnp.dot(jax.nn.relu(
            jnp.dot(q_ref[...], wc1_ref[...], preferred_element_type=jnp.float32)
            + bc1_ref[...]), wc2_ref[...], preferred_element_type=jnp.float32)
        + bc2_ref[...])


def _readout(hv, p8, q8, mid, wfp, wahv, wap8, waq8, wc18, bc1, wc2, bc38):
    return pl.pallas_call(
        _readout_body,
        in_specs=[
            _full_spec((N, HV)), _full_spec((N, 16)), _full_spec((N, 16)),
            _full_spec((N, 1)), _full_spec((HV, MM)), _full_spec((HV, 1)),
            _full_spec((16, 1)), _full_spec((16, 1)),
            _full_spec((16, MM)), _full_spec((1, MM)),
            _full_spec((MM, 8)), _full_spec((1, 8)),
        ],
        out_specs=[_full_spec((NM, MM)), _full_spec((N, 8))],
        out_shape=[
            jax.ShapeDtypeStruct((NM, MM), jnp.float32),
            jax.ShapeDtypeStruct((N, 8), jnp.float32),
        ],
    )(hv, p8, q8, mid, wfp, wahv, wap8, waq8, wc18, bc1, wc2, bc38)


_MESH = plsc.VectorSubcoreMesh(core_axis_name="c", subcore_axis_name="s")
_MESH1 = plsc.VectorSubcoreMesh(core_axis_name="c", subcore_axis_name="s",
                                num_cores=1)


def _copy_idx_row(src2d, j, dst1d):
    for k in range(CHUNK // 16):
        sl = pl.ds(k * 16, 16)
        dst1d[sl] = src2d[j, sl]


HME = ME // 2


@functools.lru_cache(maxsize=None)
def _sc_edge_msg_fn():
    out_type = [
        jax.ShapeDtypeStruct((E2, HME), jnp.float32),
        jax.ShapeDtypeStruct((E2, HME), jnp.float32),
        jax.ShapeDtypeStruct((E2, 16), jnp.float32),
        jax.ShapeDtypeStruct((2, NP, HME), jnp.float32),
        jax.ShapeDtypeStruct((2, NP, HME), jnp.float32),
    ]

    def body(tu_h, tv_h, mhe_h, ce_h, u2_h, v2_h, z_h,
             ma_out, mb_out, coef_out, agga_out, aggb_out,
             uix, vix, u1, v1, au, bv, mhv, cev, mva, mvb, cfv, shared,
             sg1, sg2, sl_):
        cid = lax.axis_index("c")
        sid = lax.axis_index("s")
        wid = cid * 16 + sid
        stripe = pl.ds(sid * STRIPE, STRIPE)
        pltpu.sync_copy(z_h.at[stripe], shared.at[stripe])
        pltpu.sync_copy(u2_h.at[pl.ds(wid * NCH, NCH)], uix)
        pltpu.sync_copy(v2_h.at[pl.ds(wid * NCH, NCH)], vix)
        plsc.subcore_barrier()

        def fire(b, j):
            base = wid * (NCH * CHUNK) + j * CHUNK
            _copy_idx_row(uix, j, u1[b])
            _copy_idx_row(vix, j, v1[b])
            pltpu.async_copy(tu_h.at[u1[b]], au[b], sg1[b])
            pltpu.async_copy(tv_h.at[v1[b]], bv[b], sg2[b])
            pltpu.async_copy(mhe_h.at[pl.ds(base, CHUNK)], mhv[b], sl_[b])
            pltpu.async_copy(ce_h.at[pl.ds(base, CHUNK)], cev[b], sl_[b])

        def process(b, j, do_fire):
            pltpu.make_async_copy(tu_h.at[pl.ds(0, CHUNK)], au[b],
                                  sg1[b]).wait()
            pltpu.make_async_copy(tv_h.at[pl.ds(0, CHUNK)], bv[b],
                                  sg2[b]).wait()
            pltpu.make_async_copy(mhe_h.at[pl.ds(0, CHUNK)], mhv[b],
                                  sl_[b]).wait()
            pltpu.make_async_copy(ce_h.at[pl.ds(0, CHUNK)], cev[b],
                                  sl_[b]).wait()

            def row(r2, c2):
                for rr in range(2):
                    r = r2 * 2 + rr
                    for cc in range(2):
                        sl = pl.ds(cc * 16, 16)
                        mva[b][r, sl] = jnp.maximum(
                            au[b][r, sl] + bv[b][r, sl] + mhv[b][r, sl], 0.0)
                    for cc in range(2):
                        sl = pl.ds(HME + cc * 16, 16)
                        so = pl.ds(cc * 16, 16)
                        mvb[b][r, so] = jnp.maximum(
                            au[b][r, sl] + bv[b][r, sl] + mhv[b][r, sl], 0.0)
                    tl = pl.ds(ME, 16)
                    s = au[b][r, tl] + bv[b][r, tl] + cev[b][r, pl.ds(0, 16)]
                    cfv[b][r, pl.ds(0, 16)] = (
                        1.0 - 2.0 / (jnp.exp(2.0 * s) + 1.0))
                return c2

            lax.fori_loop(0, CHUNK // 2, row, 0)
            pltpu.sync_copy(mva[b], shared.at[v1[b]], add=True)
            if do_fire:
                fire(b, j + 2)
            base = wid * (NCH * CHUNK) + j * CHUNK
            pltpu.sync_copy(mva[b], ma_out.at[pl.ds(base, CHUNK)])
            pltpu.sync_copy(mvb[b], mb_out.at[pl.ds(base, CHUNK)])
            pltpu.sync_copy(cfv[b], coef_out.at[pl.ds(base, CHUNK)])

        fire(0, 0)
        fire(1, 1)

        def pair(jj, carry):
            process(0, 2 * jj, True)
            process(1, 2 * jj + 1, True)
            return carry

        lax.fori_loop(0, (NCH - 2) // 2, pair, 0)
        process(0, NCH - 2, False)
        process(1, NCH - 1, False)
        plsc.subcore_barrier()
        pltpu.sync_copy(shared.at[stripe], agga_out.at[cid, stripe])
        pltpu.sync_copy(z_h.at[stripe], shared.at[stripe])
        plsc.subcore_barrier()

        def fire_b(b, j):
            base = wid * (NCH * CHUNK) + j * CHUNK
            _copy_idx_row(vix, j, v1[b])
            pltpu.async_copy(mb_out.at[pl.ds(base, CHUNK)], mvb[b], sl_[b])

        def process_b(b, j, do_fire):
            pltpu.make_async_copy(mb_out.at[pl.ds(0, CHUNK)], mvb[b],
                                  sl_[b]).wait()
            pltpu.sync_copy(mvb[b], shared.at[v1[b]], add=True)
            if do_fire:
                fire_b(b, j + 2)

        fire_b(0, 0)
        fire_b(1, 1)

        def pair_b(jj, carry):
            process_b(0, 2 * jj, True)
            process_b(1, 2 * jj + 1, True)
            return carry

        lax.fori_loop(0, (NCH - 2) // 2, pair_b, 0)
        process_b(0, NCH - 2, False)
        process_b(1, NCH - 1, False)
        plsc.subcore_barrier()
        pltpu.sync_copy(shared.at[stripe], aggb_out.at[cid, stripe])

    scratch = [
        pltpu.VMEM((NCH, CHUNK), jnp.int32),
        pltpu.VMEM((NCH, CHUNK), jnp.int32),
        [pltpu.VMEM((CHUNK,), jnp.int32)] * 2,
        [pltpu.VMEM((CHUNK,), jnp.int32)] * 2,
        [pltpu.VMEM((CHUNK, 80), jnp.float32)] * 2,
        [pltpu.VMEM((CHUNK, 80), jnp.float32)] * 2,
        [pltpu.VMEM((CHUNK, ME), jnp.float32)] * 2,
        [pltpu.VMEM((CHUNK, 16), jnp.float32)] * 2,
        [pltpu.VMEM((CHUNK, HME), jnp.float32)] * 2,
        [pltpu.VMEM((CHUNK, HME), jnp.float32)] * 2,
        [pltpu.VMEM((CHUNK, 16), jnp.float32)] * 2,
        pltpu.VMEM_SHARED((NP, HME), jnp.float32),
        [pltpu.SemaphoreType.DMA] * 2,
        [pltpu.SemaphoreType.DMA] * 2,
        [pltpu.SemaphoreType.DMA] * 2,
    ]
    return pl.kernel(body, out_type=out_type, mesh=_MESH,
                     scratch_types=scratch,
                     compiler_params=pltpu.CompilerParams(
                         use_tc_tiling_on_sc=False))


def _sc_edge_msg(tu, tv, mhe, ce, u2d, v2d, zeros32):
    return _sc_edge_msg_fn()(tu, tv, mhe, ce, u2d, v2d, zeros32)


NCH_H = E2 // (16 * CHUNK)


@functools.lru_cache(maxsize=None)
def _sc_ham_f_fn():
    out_type = [jax.ShapeDtypeStruct((2, NP, 16), jnp.float32)]

    def body(q_h, cf_h, u2_h, v2_h, z_h, f_out, uix, vix, u1, v1,
             qu, qv, wv, cfv, shared, sg1, sg2, sl_):
        cid = lax.axis_index("c")
        sid = lax.axis_index("s")
        wid = cid * 16 + sid
        stripe = pl.ds(sid * STRIPE, STRIPE)
        pltpu.sync_copy(z_h.at[stripe], shared.at[stripe])
        pltpu.sync_copy(u2_h.at[pl.ds(wid * NCH, NCH)], uix)
        pltpu.sync_copy(v2_h.at[pl.ds(wid * NCH, NCH)], vix)
        plsc.subcore_barrier()

        def fire(b, j):
            base = wid * (NCH * CHUNK) + j * CHUNK
            _copy_idx_row(uix, j, u1[b])
            _copy_idx_row(vix, j, v1[b])
            pltpu.async_copy(q_h.at[u1[b]], qu[b], sg1[b])
            pltpu.async_copy(q_h.at[v1[b]], qv[b], sg2[b])
            pltpu.async_copy(cf_h.at[pl.ds(base, CHUNK)], cfv[b], sl_[b])

        def process(b, j, do_fire):
            pltpu.make_async_copy(q_h.at[pl.ds(0, CHUNK)], qu[b],
                                  sg1[b]).wait()
            pltpu.make_async_copy(q_h.at[pl.ds(0, CHUNK)], qv[b],
                                  sg2[b]).wait()
            pltpu.make_async_copy(cf_h.at[pl.ds(0, CHUNK)], cfv[b],
                                  sl_[b]).wait()

            def row(r4, c2):
                for rr in range(4):
                    r = r4 * 4 + rr
                    sl = pl.ds(0, 16)
                    wv[b][r, sl] = cfv[b][r, sl] * (qv[b][r, sl] - qu[b][r, sl])
                return c2

            lax.fori_loop(0, CHUNK // 4, row, 0)
            pltpu.sync_copy(wv[b], shared.at[u1[b]], add=True)
            if do_fire:
                fire(b, j + 2)

        fire(0, 0)
        fire(1, 1)

        def pair(jj, carry):
            process(0, 2 * jj, True)
            process(1, 2 * jj + 1, True)
            return carry

        lax.fori_loop(0, (NCH - 2) // 2, pair, 0)
        process(0, NCH - 2, False)
        process(1, NCH - 1, False)
        plsc.subcore_barrier()
        pltpu.sync_copy(shared.at[stripe], f_out.at[cid, stripe])

    scratch = [
        pltpu.VMEM((NCH, CHUNK), jnp.int32),
        pltpu.VMEM((NCH, CHUNK), jnp.int32),
        [pltpu.VMEM((CHUNK,), jnp.int32)] * 2,
        [pltpu.VMEM((CHUNK,), jnp.int32)] * 2,
        [pltpu.VMEM((CHUNK, 16), jnp.float32)] * 2,
        [pltpu.VMEM((CHUNK, 16), jnp.float32)] * 2,
        [pltpu.VMEM((CHUNK, 16), jnp.float32)] * 2,
        [pltpu.VMEM((CHUNK, 16), jnp.float32)] * 2,
        pltpu.VMEM_SHARED((NP, 16), jnp.float32),
        [pltpu.SemaphoreType.DMA] * 2,
        [pltpu.SemaphoreType.DMA] * 2,
        [pltpu.SemaphoreType.DMA] * 2,
    ]
    return pl.kernel(body, out_type=out_type, mesh=_MESH,
                     scratch_types=scratch,
                     compiler_params=pltpu.CompilerParams(
                         use_tc_tiling_on_sc=False))


def _ham_update_body(q_ref, p_ref, f_ref, mass_ref, qo_ref, po_ref):
    f = f_ref[0] + f_ref[1]
    qo_ref[...] = q_ref[...] + TAU * p_ref[...] / mass_ref[...]
    po_ref[...] = (1.0 - 0.1 * TAU) * p_ref[...] + TAU * f


def _ham_update(q16, p16, f2, mass):
    bn2 = NP // 4
    return pl.pallas_call(
        _ham_update_body,
        grid=(4,),
        in_specs=[
            _row_spec(bn2, 16), _row_spec(bn2, 16),
            pl.BlockSpec((2, bn2, 16), lambda i: (0, i, 0)),
            _row_spec(bn2, 1),
        ],
        out_specs=[_row_spec(bn2, 16), _row_spec(bn2, 16)],
        out_shape=[
            jax.ShapeDtypeStruct((NP, 16), jnp.float32),
            jax.ShapeDtypeStruct((NP, 16), jnp.float32),
        ],
    )(q16, p16, f2, mass)


def _sc_ham_f(q16, coef, u2d, v2d, zeros16):
    return _sc_ham_f_fn()(q16, coef, u2d, v2d, zeros16)[0]


def kernel(atom_ftr, bond_ftr, massive, edge_index, mol_ids,
           W_init_v, b_init_v, W_init_e, b_init_e, W_p, W_q, W_msg,
           W_upd_v, W_upd_e, W_ham, W_att, W_fp, W_c1, b_c1, W_c2, b_c2):
    def pad_cols(w, cols):
        return jnp.pad(w, ((0, 0), (0, cols - w.shape[1])))

    def pad_rows(w, rows):
        return jnp.pad(w, ((0, rows - w.shape[0]), (0, 0)))

    def pad_nodes(x):
        return jnp.pad(x, ((0, NP - N), (0, 0)))

    def pad_edges(x):
        return jnp.pad(x, ((0, E2 - E), (0, 0)))

    u2d = jnp.concatenate(
        [edge_index[0].astype(jnp.int32),
         jnp.full((E2 - E,), N, jnp.int32)]).reshape(E2 // CHUNK, CHUNK)
    v2d = jnp.concatenate(
        [edge_index[1].astype(jnp.int32),
         jnp.full((E2 - E,), N, jnp.int32)]).reshape(E2 // CHUNK, CHUNK)
    zeros32 = jnp.zeros((NP, HME), jnp.float32)
    zeros16 = jnp.zeros((NP, 16), jnp.float32)
    mass_pad = jnp.pad(massive, ((0, NP - N), (0, 0)), constant_values=1.0)

    wp16 = pad_cols(W_p, 16)
    wq16 = pad_cols(W_q, 16)
    wd16 = [pad_rows(W_msg[i][2 * HV + HE:], 16) for i in range(2)]
    wmu = [W_msg[i][:HV] for i in range(2)]
    wmv = [W_msg[i][HV:2 * HV] for i in range(2)]
    whu = [W_ham[i][:HV] for i in range(2)]
    whv = [W_ham[i][HV:2 * HV] for i in range(2)]

    hv, p16, q16, tu, tv = _init_nodes(
        atom_ftr, W_init_v, b_init_v[None], wp16, wq16,
        wmu[0], wmv[0], wd16[0], whu[0], whv[0])
    p16 = pad_nodes(p16)
    q16 = pad_nodes(q16)
    he, mhe, ce = _init_edges(
        bond_ftr, W_init_e, b_init_e[None],
        W_msg[0][2 * HV:2 * HV + HE], W_ham[0][2 * HV:])

    nxt = [1, 1]
    xs = dict(
        wuv1=jnp.stack([W_upd_v[i][:HV] for i in range(2)]),
        wuv2=jnp.stack([W_upd_v[i][HV:] for i in range(2)]),
        wa=jnp.stack([W_upd_e[i][:HE] for i in range(2)]),
        wb=jnp.stack([W_upd_e[i][HE:] for i in range(2)]),
        wc=jnp.stack([W_msg[j][2 * HV:2 * HV + HE] for j in nxt]),
        wdh=jnp.stack([W_ham[j][2 * HV:] for j in nxt]),
        wmu_n=jnp.stack([wmu[j] for j in nxt]),
        wmv_n=jnp.stack([wmv[j] for j in nxt]),
        wd16_n=jnp.stack([wd16[j] for j in nxt]),
        whu_n=jnp.stack([whu[j] for j in nxt]),
        whv_n=jnp.stack([whv[j] for j in nxt]),
    )

    def layer_body(carry, ws):
        hv, q16, p16, tu, tv, he, mhe, ce = carry
        ma, mb, coef, agga, aggb = _sc_edge_msg(
            pad_nodes(tu), pad_nodes(tv), pad_edges(mhe), pad_edges(ce),
            u2d, v2d, zeros32)
        t_hv = _upd_v(hv, agga[:, :N], aggb[:, :N], ws['wuv1'],
                      ws['wuv2'][:HME], ws['wuv2'][HME:])
        t_he, mhe_n, ce_n = _upd_e_proj(he, ma[:E], mb[:E], ws['wa'],
                                        ws['wb'][:HME], ws['wb'][HME:],
                                        ws['wc'], ws['wdh'])
        for _ in range(4):
            f2 = _sc_ham_f(q16, coef, u2d, v2d, zeros16)
            q16, p16 = _ham_update(q16, p16, f2, mass_pad)
        tu_n, tv_n = _node_tables(t_hv, q16[:N], ws['wmu_n'], ws['wmv_n'],
                                  ws['wd16_n'], ws['whu_n'], ws['whv_n'])
        return (t_hv, q16, p16, tu_n, tv_n, t_he, mhe_n, ce_n), None

    carry, _ = lax.scan(layer_body, (hv, q16, p16, tu, tv, he, mhe, ce), xs,
                        length=2)
    hv, q16, p16 = carry[0], carry[1], carry[2]

    fp, conf8 = _readout(
        hv, p16[:N], q16[:N], mol_ids[:, None].astype(jnp.int32),
        W_fp, W_att[:HV], pad_rows(W_att[HV:HV + PQ], 16),
        pad_rows(W_att[HV + PQ:], 16), pad_rows(W_c1, 16), b_c1[None],
        pad_cols(W_c2, 8), pad_cols(b_c2[None], 8))
    return (fp, conf8[:, :PQ])


# speedup vs baseline: 1.1409x; 1.0316x over previous
import functools
import jax
import jax.numpy as jnp
from jax import lax
from jax.experimental import pallas as pl
from jax.experimental.pallas import tpu as pltpu
from jax.experimental.pallas import tpu_sc as plsc

N = 10000
E = 160000
HV = 128
HE = 64
PQ = 3
ME = 64
MM = 128
NM = 64
TAU = 0.25

BN = 1000
BE = 2000

NP = 10112
E2 = 163840
NW = 32
CHUNK = 128
NCH = E2 // (NW * CHUNK)
STRIPE = NP // 16


def _row_spec(block_rows, cols):
    return pl.BlockSpec((block_rows, cols), lambda i: (i, 0))


def _full_spec(shape):
    return pl.BlockSpec(shape, lambda *a: tuple(0 for _ in shape))


def _init_nodes_body(atom_ref, wiv_ref, biv_ref, wp_ref, wq_ref,
                     wmu_ref, wmv_ref, wd_ref, whu_ref, whv_ref,
                     hv_ref, p_ref, q_ref, tu_ref, tv_ref):
    hv = jnp.tanh(jnp.dot(atom_ref[...], wiv_ref[...],
                          preferred_element_type=jnp.float32) + biv_ref[...])
    hv_ref[...] = hv
    p = jnp.dot(hv, wp_ref[...], preferred_element_type=jnp.float32)
    q = jnp.dot(hv, wq_ref[...], preferred_element_type=jnp.float32)
    p_ref[...] = p
    q_ref[...] = q
    qd = jnp.dot(q, wd_ref[...], preferred_element_type=jnp.float32)
    a = jnp.dot(hv, wmu_ref[...], preferred_element_type=jnp.float32) + qd
    b = jnp.dot(hv, wmv_ref[...], preferred_element_type=jnp.float32) - qd
    cu = jnp.dot(hv, whu_ref[...], preferred_element_type=jnp.float32)
    cv = jnp.dot(hv, whv_ref[...], preferred_element_type=jnp.float32)
    tu_ref[...] = jnp.concatenate(
        [a, jnp.broadcast_to(cu, (a.shape[0], 16))], axis=1)
    tv_ref[...] = jnp.concatenate(
        [b, jnp.broadcast_to(cv, (b.shape[0], 16))], axis=1)


def _init_nodes(atom_ftr, wiv, biv, wp8, wq8, wmu, wmv, wd8, whu, whv):
    return pl.pallas_call(
        _init_nodes_body,
        grid=(N // BN,),
        in_specs=[
            _row_spec(BN, HV), _full_spec((HV, HV)), _full_spec((1, HV)),
            _full_spec((HV, 16)), _full_spec((HV, 16)),
            _full_spec((HV, ME)), _full_spec((HV, ME)), _full_spec((16, ME)),
            _full_spec((HV, 1)), _full_spec((HV, 1)),
        ],
        out_specs=[
            _row_spec(BN, HV), _row_spec(BN, 16), _row_spec(BN, 16),
            _row_spec(BN, 80), _row_spec(BN, 80),
        ],
        out_shape=[
            jax.ShapeDtypeStruct((N, HV), jnp.float32),
            jax.ShapeDtypeStruct((N, 16), jnp.float32),
            jax.ShapeDtypeStruct((N, 16), jnp.float32),
            jax.ShapeDtypeStruct((N, 80), jnp.float32),
            jax.ShapeDtypeStruct((N, 80), jnp.float32),
        ],
    )(atom_ftr, wiv, biv, wp8, wq8, wmu, wmv, wd8, whu, whv)


def _node_tables_body(hv_ref, q_ref, wmu_ref, wmv_ref, wd_ref, whu_ref,
                      whv_ref, tu_ref, tv_ref):
    hv = hv_ref[...]
    qd = jnp.dot(q_ref[...], wd_ref[...], preferred_element_type=jnp.float32)
    a = jnp.dot(hv, wmu_ref[...], preferred_element_type=jnp.float32) + qd
    b = jnp.dot(hv, wmv_ref[...], preferred_element_type=jnp.float32) - qd
    cu = jnp.dot(hv, whu_ref[...], preferred_element_type=jnp.float32)
    cv = jnp.dot(hv, whv_ref[...], preferred_element_type=jnp.float32)
    tu_ref[...] = jnp.concatenate(
        [a, jnp.broadcast_to(cu, (a.shape[0], 16))], axis=1)
    tv_ref[...] = jnp.concatenate(
        [b, jnp.broadcast_to(cv, (b.shape[0], 16))], axis=1)


def _node_tables(hv, q8, wmu, wmv, wd8, whu, whv):
    return pl.pallas_call(
        _node_tables_body,
        grid=(N // BN,),
        in_specs=[
            _row_spec(BN, HV), _row_spec(BN, 16),
            _full_spec((HV, ME)), _full_spec((HV, ME)), _full_spec((16, ME)),
            _full_spec((HV, 1)), _full_spec((HV, 1)),
        ],
        out_specs=[_row_spec(BN, 80), _row_spec(BN, 80)],
        out_shape=[
            jax.ShapeDtypeStruct((N, 80), jnp.float32),
            jax.ShapeDtypeStruct((N, 80), jnp.float32),
        ],
    )(hv, q8, wmu, wmv, wd8, whu, whv)


def _init_edges_body(bond_ref, wie_ref, bie_ref, wme_ref, whe_ref,
                     he_ref, mhe_ref, ce_ref):
    he = jnp.tanh(jnp.dot(bond_ref[...], wie_ref[...],
                          preferred_element_type=jnp.float32) + bie_ref[...])
    he_ref[...] = he
    mhe_ref[...] = jnp.dot(he, wme_ref[...], preferred_element_type=jnp.float32)
    ce = jnp.dot(he, whe_ref[...], preferred_element_type=jnp.float32)
    ce_ref[...] = jnp.broadcast_to(ce, (ce.shape[0], 16))


def _init_edges(bond_ftr, wie, bie, wme, whe):
    return pl.pallas_call(
        _init_edges_body,
        grid=(E // BE,),
        in_specs=[
            _row_spec(BE, 16), _full_spec((16, HE)), _full_spec((1, HE)),
            _full_spec((HE, ME)), _full_spec((HE, 1)),
        ],
        out_specs=[_row_spec(BE, HE), _row_spec(BE, ME), _row_spec(BE, 16)],
        out_shape=[
            jax.ShapeDtypeStruct((E, HE), jnp.float32),
            jax.ShapeDtypeStruct((E, ME), jnp.float32),
            jax.ShapeDtypeStruct((E, 16), jnp.float32),
        ],
    )(bond_ftr, wie, bie, wme, whe)


def _upd_v_body(hv_ref, agga_ref, aggb_ref, wv1_ref, wv2a_ref, wv2b_ref,
                out_ref):
    out_ref[...] = jax.nn.relu(
        jnp.dot(hv_ref[...], wv1_ref[...], preferred_element_type=jnp.float32)
        + jnp.dot(agga_ref[0] + agga_ref[1], wv2a_ref[...],
                  preferred_element_type=jnp.float32)
        + jnp.dot(aggb_ref[0] + aggb_ref[1], wv2b_ref[...],
                  preferred_element_type=jnp.float32))


def _upd_v(hv, agga, aggb, wv1, wv2a, wv2b):
    return pl.pallas_call(
        _upd_v_body,
        grid=(N // BN,),
        in_specs=[
            _row_spec(BN, HV),
            pl.BlockSpec((2, BN, HME), lambda i: (0, i, 0)),
            pl.BlockSpec((2, BN, HME), lambda i: (0, i, 0)),
            _full_spec((HV, HV)), _full_spec((HME, HV)),
            _full_spec((HME, HV)),
        ],
        out_specs=_row_spec(BN, HV),
        out_shape=jax.ShapeDtypeStruct((N, HV), jnp.float32),
    )(hv, agga, aggb, wv1, wv2a, wv2b)


def _upd_e_proj_body(he_ref, ma_ref, mb_ref, wa_ref, wb1_ref, wb2_ref,
                     wc_ref, wd_ref, the_ref, mhe_ref, ce_ref):
    t_he = jax.nn.relu(
        jnp.dot(he_ref[...], wa_ref[...], preferred_element_type=jnp.float32)
        + jnp.dot(ma_ref[...], wb1_ref[...],
                  preferred_element_type=jnp.float32)
        + jnp.dot(mb_ref[...], wb2_ref[...],
                  preferred_element_type=jnp.float32))
    the_ref[...] = t_he
    mhe_ref[...] = jnp.dot(t_he, wc_ref[...], preferred_element_type=jnp.float32)
    ce = jnp.dot(t_he, wd_ref[...], preferred_element_type=jnp.float32)
    ce_ref[...] = jnp.broadcast_to(ce, (ce.shape[0], 16))


def _upd_e_proj(he, ma, mb, wa, wb1, wb2, wc, wd):
    return pl.pallas_call(
        _upd_e_proj_body,
        grid=(E // BE,),
        in_specs=[
            _row_spec(BE, HE), _row_spec(BE, HME), _row_spec(BE, HME),
            _full_spec((HE, HE)), _full_spec((HME, HE)),
            _full_spec((HME, HE)),
            _full_spec((HE, ME)), _full_spec((HE, 1)),
        ],
        out_specs=[_row_spec(BE, HE), _row_spec(BE, ME), _row_spec(BE, 16)],
        out_shape=[
            jax.ShapeDtypeStruct((E, HE), jnp.float32),
            jax.ShapeDtypeStruct((E, ME), jnp.float32),
            jax.ShapeDtypeStruct((E, 16), jnp.float32),
        ],
    )(he, ma, mb, wa, wb1, wb2, wc, wd)


def _readout_body(hv_ref, p_ref, q_ref, mid_ref, wfp_ref, wahv_ref, wap_ref,
                  waq_ref, wc1_ref, bc1_ref, wc2_ref, bc2_ref,
                  fp_ref, conf_ref):
    hv = hv_ref[...]
    hvp = jnp.dot(hv, wfp_ref[...], preferred_element_type=jnp.float32)
    a = jax.nn.sigmoid(
        jnp.dot(hv, wahv_ref[...], preferred_element_type=jnp.float32)
        + jnp.dot(p_ref[...], wap_ref[...], preferred_element_type=jnp.float32)
        + jnp.dot(q_ref[...], waq_ref[...], preferred_element_type=jnp.float32))
    mids = mid_ref[...]
    oh = (mids == lax.broadcasted_iota(jnp.int32, (N, NM), 1)).astype(jnp.float32)
    hm = lax.dot_general(oh, a * hvp, (((0,), (0,)), ((), ())),
                         preferred_element_type=jnp.float32)
    for _ in range(2):
        g = jnp.dot(oh, hm, preferred_element_type=jnp.float32)
        a2 = jax.nn.sigmoid(jnp.sum(hvp * g, axis=1, keepdims=True))
        hm = lax.dot_general(oh, a2 * hvp, (((0,), (0,)), ((), ())),
                             preferred_element_type=jnp.float32)
    fp_ref[...] = hm
    conf_ref[...] = (
        jnp.dot(jax.nn.relu(
            jnp.dot(q_ref[...], wc1_ref[...], preferred_element_type=jnp.float32)
            + bc1_ref[...]), wc2_ref[...], preferred_element_type=jnp.float32)
        + bc2_ref[...])


def _readout(hv, p8, q8, mid, wfp, wahv, wap8, waq8, wc18, bc1, wc2, bc38):
    return pl.pallas_call(
        _readout_body,
        in_specs=[
            _full_spec((N, HV)), _full_spec((N, 16)), _full_spec((N, 16)),
            _full_spec((N, 1)), _full_spec((HV, MM)), _full_spec((HV, 1)),
            _full_spec((16, 1)), _full_spec((16, 1)),
            _full_spec((16, MM)), _full_spec((1, MM)),
            _full_spec((MM, 8)), _full_spec((1, 8)),
        ],
        out_specs=[_full_spec((NM, MM)), _full_spec((N, 8))],
        out_shape=[
            jax.ShapeDtypeStruct((NM, MM), jnp.float32),
            jax.ShapeDtypeStruct((N, 8), jnp.float32),
        ],
    )(hv, p8, q8, mid, wfp, wahv, wap8, waq8, wc18, bc1, wc2, bc38)


_MESH = plsc.VectorSubcoreMesh(core_axis_name="c", subcore_axis_name="s")
_MESH1 = plsc.VectorSubcoreMesh(core_axis_name="c", subcore_axis_name="s",
                                num_cores=1)


def _copy_idx_row(src2d, j, dst1d):
    for k in range(CHUNK // 16):
        sl = pl.ds(k * 16, 16)
        dst1d[sl] = src2d[j, sl]


HME = ME // 2


@functools.lru_cache(maxsize=None)
def _sc_edge_msg_fn():
    out_type = [
        jax.ShapeDtypeStruct((E2, HME), jnp.float32),
        jax.ShapeDtypeStruct((E2, HME), jnp.float32),
        jax.ShapeDtypeStruct((E2, 16), jnp.float32),
        jax.ShapeDtypeStruct((2, NP, HME), jnp.float32),
        jax.ShapeDtypeStruct((2, NP, HME), jnp.float32),
    ]

    def body(tu_h, tv_h, mhe_h, ce_h, u2_h, v2_h, z_h,
             ma_out, mb_out, coef_out, agga_out, aggb_out,
             uix, vix, u1, v1, au, bv, mhv, cev, mva, mvb, cfv, shared,
             sg1, sg2, sl_):
        cid = lax.axis_index("c")
        sid = lax.axis_index("s")
        wid = cid * 16 + sid
        stripe = pl.ds(sid * STRIPE, STRIPE)
        pltpu.sync_copy(z_h.at[stripe], shared.at[stripe])
        pltpu.sync_copy(u2_h.at[pl.ds(wid * NCH, NCH)], uix)
        pltpu.sync_copy(v2_h.at[pl.ds(wid * NCH, NCH)], vix)
        plsc.subcore_barrier()

        def fire(b, j):
            base = wid * (NCH * CHUNK) + j * CHUNK
            _copy_idx_row(uix, j, u1[b])
            _copy_idx_row(vix, j, v1[b])
            pltpu.async_copy(tu_h.at[u1[b]], au[b], sg1[b])
            pltpu.async_copy(tv_h.at[v1[b]], bv[b], sg2[b])
            pltpu.async_copy(mhe_h.at[pl.ds(base, CHUNK)], mhv[b], sl_[b])
            pltpu.async_copy(ce_h.at[pl.ds(base, CHUNK)], cev[b], sl_[b])

        def process(b, j, do_fire):
            pltpu.make_async_copy(tu_h.at[pl.ds(0, CHUNK)], au[b],
                                  sg1[b]).wait()
            pltpu.make_async_copy(tv_h.at[pl.ds(0, CHUNK)], bv[b],
                                  sg2[b]).wait()
            pltpu.make_async_copy(mhe_h.at[pl.ds(0, CHUNK)], mhv[b],
                                  sl_[b]).wait()
            pltpu.make_async_copy(ce_h.at[pl.ds(0, CHUNK)], cev[b],
                                  sl_[b]).wait()

            def row(r2, c2):
                for rr in range(2):
                    r = r2 * 2 + rr
                    for cc in range(2):
                        sl = pl.ds(cc * 16, 16)
                        mva[b][r, sl] = jnp.maximum(
                            au[b][r, sl] + bv[b][r, sl] + mhv[b][r, sl], 0.0)
                    for cc in range(2):
                        sl = pl.ds(HME + cc * 16, 16)
                        so = pl.ds(cc * 16, 16)
                        mvb[b][r, so] = jnp.maximum(
                            au[b][r, sl] + bv[b][r, sl] + mhv[b][r, sl], 0.0)
                    tl = pl.ds(ME, 16)
                    s = au[b][r, tl] + bv[b][r, tl] + cev[b][r, pl.ds(0, 16)]
                    cfv[b][r, pl.ds(0, 16)] = (
                        1.0 - 2.0 / (jnp.exp(2.0 * s) + 1.0))
                return c2

            lax.fori_loop(0, CHUNK // 2, row, 0)
            pltpu.sync_copy(mva[b], shared.at[v1[b]], add=True)
            if do_fire:
                fire(b, j + 2)
            base = wid * (NCH * CHUNK) + j * CHUNK
            pltpu.sync_copy(mva[b], ma_out.at[pl.ds(base, CHUNK)])
            pltpu.sync_copy(mvb[b], mb_out.at[pl.ds(base, CHUNK)])
            pltpu.sync_copy(cfv[b], coef_out.at[pl.ds(base, CHUNK)])

        fire(0, 0)
        fire(1, 1)

        def pair(jj, carry):
            process(0, 2 * jj, True)
            process(1, 2 * jj + 1, True)
            return carry

        lax.fori_loop(0, (NCH - 2) // 2, pair, 0)
        process(0, NCH - 2, False)
        process(1, NCH - 1, False)
        plsc.subcore_barrier()
        pltpu.sync_copy(shared.at[stripe], agga_out.at[cid, stripe])
        pltpu.sync_copy(z_h.at[stripe], shared.at[stripe])
        plsc.subcore_barrier()

        def fire_b(b, j):
            base = wid * (NCH * CHUNK) + j * CHUNK
            _copy_idx_row(vix, j, v1[b])
            pltpu.async_copy(mb_out.at[pl.ds(base, CHUNK)], mvb[b], sl_[b])

        def process_b(b, j, do_fire):
            pltpu.make_async_copy(mb_out.at[pl.ds(0, CHUNK)], mvb[b],
                                  sl_[b]).wait()
            pltpu.sync_copy(mvb[b], shared.at[v1[b]], add=True)
            if do_fire:
                fire_b(b, j + 2)

        fire_b(0, 0)
        fire_b(1, 1)

        def pair_b(jj, carry):
            process_b(0, 2 * jj, True)
            process_b(1, 2 * jj + 1, True)
            return carry

        lax.fori_loop(0, (NCH - 2) // 2, pair_b, 0)
        process_b(0, NCH - 2, False)
        process_b(1, NCH - 1, False)
        plsc.subcore_barrier()
        pltpu.sync_copy(shared.at[stripe], aggb_out.at[cid, stripe])

    scratch = [
        pltpu.VMEM((NCH, CHUNK), jnp.int32),
        pltpu.VMEM((NCH, CHUNK), jnp.int32),
        [pltpu.VMEM((CHUNK,), jnp.int32)] * 2,
        [pltpu.VMEM((CHUNK,), jnp.int32)] * 2,
        [pltpu.VMEM((CHUNK, 80), jnp.float32)] * 2,
        [pltpu.VMEM((CHUNK, 80), jnp.float32)] * 2,
        [pltpu.VMEM((CHUNK, ME), jnp.float32)] * 2,
        [pltpu.VMEM((CHUNK, 16), jnp.float32)] * 2,
        [pltpu.VMEM((CHUNK, HME), jnp.float32)] * 2,
        [pltpu.VMEM((CHUNK, HME), jnp.float32)] * 2,
        [pltpu.VMEM((CHUNK, 16), jnp.float32)] * 2,
        pltpu.VMEM_SHARED((NP, HME), jnp.float32),
        [pltpu.SemaphoreType.DMA] * 2,
        [pltpu.SemaphoreType.DMA] * 2,
        [pltpu.SemaphoreType.DMA] * 2,
    ]
    return pl.kernel(body, out_type=out_type, mesh=_MESH,
                     scratch_types=scratch,
                     compiler_params=pltpu.CompilerParams(
                         use_tc_tiling_on_sc=False))


def _sc_edge_msg(tu, tv, mhe, ce, u2d, v2d, zeros32):
    return _sc_edge_msg_fn()(tu, tv, mhe, ce, u2d, v2d, zeros32)


NCH_H = E2 // (16 * CHUNK)


@functools.lru_cache(maxsize=None)
def _sc_ham_f_fn():
    out_type = [jax.ShapeDtypeStruct((2, NP, 32), jnp.float32)]

    def body(q_h, cf_h, u2_h, v2_h, z_h, f_out, uix, vix, u1, v1,
             qu, qv, wv, cfv, shared, sg1, sg2, sl_):
        cid = lax.axis_index("c")
        sid = lax.axis_index("s")
        wid = cid * 16 + sid
        stripe = pl.ds(sid * STRIPE, STRIPE)
        pltpu.sync_copy(z_h.at[stripe], shared.at[stripe])
        pltpu.sync_copy(u2_h.at[pl.ds(wid * NCH, NCH)], uix)
        pltpu.sync_copy(v2_h.at[pl.ds(wid * NCH, NCH)], vix)
        plsc.subcore_barrier()

        def fire(b, j):
            base = wid * (NCH * CHUNK) + j * CHUNK
            _copy_idx_row(uix, j, u1[b])
            _copy_idx_row(vix, j, v1[b])
            pltpu.async_copy(q_h.at[u1[b]], qu[b], sg1[b])
            pltpu.async_copy(q_h.at[v1[b]], qv[b], sg2[b])
            pltpu.async_copy(cf_h.at[pl.ds(base, CHUNK)], cfv[b], sl_[b])

        def process(b, j, do_fire):
            pltpu.make_async_copy(q_h.at[pl.ds(0, CHUNK)], qu[b],
                                  sg1[b]).wait()
            pltpu.make_async_copy(q_h.at[pl.ds(0, CHUNK)], qv[b],
                                  sg2[b]).wait()
            pltpu.make_async_copy(cf_h.at[pl.ds(0, CHUNK)], cfv[b],
                                  sl_[b]).wait()

            def row(r4, c2):
                for rr in range(4):
                    r = r4 * 4 + rr
                    cf = cfv[b][r, pl.ds(0, 16)]
                    for half in (0, 16):
                        sl = pl.ds(half, 16)
                        wv[b][r, sl] = cf * (qv[b][r, sl] - qu[b][r, sl])
                return c2

            lax.fori_loop(0, CHUNK // 4, row, 0)
            pltpu.sync_copy(wv[b], shared.at[u1[b]], add=True)
            if do_fire:
                fire(b, j + 2)

        fire(0, 0)
        fire(1, 1)

        def pair(jj, carry):
            process(0, 2 * jj, True)
            process(1, 2 * jj + 1, True)
            return carry

        lax.fori_loop(0, (NCH - 2) // 2, pair, 0)
        process(0, NCH - 2, False)
        process(1, NCH - 1, False)
        plsc.subcore_barrier()
        pltpu.sync_copy(shared.at[stripe], f_out.at[cid, stripe])

    scratch = [
        pltpu.VMEM((NCH, CHUNK), jnp.int32),
        pltpu.VMEM((NCH, CHUNK), jnp.int32),
        [pltpu.VMEM((CHUNK,), jnp.int32)] * 2,
        [pltpu.VMEM((CHUNK,), jnp.int32)] * 2,
        [pltpu.VMEM((CHUNK, 32), jnp.float32)] * 2,
        [pltpu.VMEM((CHUNK, 32), jnp.float32)] * 2,
        [pltpu.VMEM((CHUNK, 32), jnp.float32)] * 2,
        [pltpu.VMEM((CHUNK, 16), jnp.float32)] * 2,
        pltpu.VMEM_SHARED((NP, 32), jnp.float32),
        [pltpu.SemaphoreType.DMA] * 2,
        [pltpu.SemaphoreType.DMA] * 2,
        [pltpu.SemaphoreType.DMA] * 2,
    ]
    return pl.kernel(body, out_type=out_type, mesh=_MESH,
                     scratch_types=scratch,
                     compiler_params=pltpu.CompilerParams(
                         use_tc_tiling_on_sc=False))


D_ = 1.0 - 0.1 * TAU


def _ham_pre_body(q_ref, p_ref, m_ref, qq_ref):
    q1 = q_ref[...] + TAU * p_ref[...] / m_ref[...]
    qq_ref[...] = jnp.concatenate([q_ref[...], q1], axis=1)


def _ham_mid_body(qq_ref, p_ref, f_ref, m_ref, qq2_ref, p2_ref):
    f0 = f_ref[0, :, :16] + f_ref[1, :, :16]
    f1 = f_ref[0, :, 16:] + f_ref[1, :, 16:]
    q1 = qq_ref[:, 16:]
    p1 = D_ * p_ref[...] + TAU * f0
    q2 = q1 + TAU * p1 / m_ref[...]
    p2 = D_ * p1 + TAU * f1
    q3 = q2 + TAU * p2 / m_ref[...]
    qq2_ref[...] = jnp.concatenate([q2, q3], axis=1)
    p2_ref[...] = p2


def _ham_fin_body(qq2_ref, p2_ref, f_ref, m_ref, q4_ref, p4_ref):
    f2 = f_ref[0, :, :16] + f_ref[1, :, :16]
    f3 = f_ref[0, :, 16:] + f_ref[1, :, 16:]
    q3 = qq2_ref[:, 16:]
    p3 = D_ * p2_ref[...] + TAU * f2
    q4_ref[...] = q3 + TAU * p3 / m_ref[...]
    p4_ref[...] = D_ * p3 + TAU * f3


_BN2 = NP // 4


def _ham_pre(q16, p16, mass):
    return pl.pallas_call(
        _ham_pre_body,
        grid=(4,),
        in_specs=[_row_spec(_BN2, 16), _row_spec(_BN2, 16),
                  _row_spec(_BN2, 1)],
        out_specs=_row_spec(_BN2, 32),
        out_shape=jax.ShapeDtypeStruct((NP, 32), jnp.float32),
    )(q16, p16, mass)


def _ham_mid(qq, p16, f2, mass):
    return pl.pallas_call(
        _ham_mid_body,
        grid=(4,),
        in_specs=[_row_spec(_BN2, 32), _row_spec(_BN2, 16),
                  pl.BlockSpec((2, _BN2, 32), lambda i: (0, i, 0)),
                  _row_spec(_BN2, 1)],
        out_specs=[_row_spec(_BN2, 32), _row_spec(_BN2, 16)],
        out_shape=[jax.ShapeDtypeStruct((NP, 32), jnp.float32),
                   jax.ShapeDtypeStruct((NP, 16), jnp.float32)],
    )(qq, p16, f2, mass)


def _ham_fin(qq2, p2, f2, mass):
    return pl.pallas_call(
        _ham_fin_body,
        grid=(4,),
        in_specs=[_row_spec(_BN2, 32), _row_spec(_BN2, 16),
                  pl.BlockSpec((2, _BN2, 32), lambda i: (0, i, 0)),
                  _row_spec(_BN2, 1)],
        out_specs=[_row_spec(_BN2, 16), _row_spec(_BN2, 16)],
        out_shape=[jax.ShapeDtypeStruct((NP, 16), jnp.float32),
                   jax.ShapeDtypeStruct((NP, 16), jnp.float32)],
    )(qq2, p2, f2, mass)


def _sc_ham_f(qq, coef, u2d, v2d, zeros32):
    return _sc_ham_f_fn()(qq, coef, u2d, v2d, zeros32)[0]


def kernel(atom_ftr, bond_ftr, massive, edge_index, mol_ids,
           W_init_v, b_init_v, W_init_e, b_init_e, W_p, W_q, W_msg,
           W_upd_v, W_upd_e, W_ham, W_att, W_fp, W_c1, b_c1, W_c2, b_c2):
    def pad_cols(w, cols):
        return jnp.pad(w, ((0, 0), (0, cols - w.shape[1])))

    def pad_rows(w, rows):
        return jnp.pad(w, ((0, rows - w.shape[0]), (0, 0)))

    def pad_nodes(x):
        return jnp.pad(x, ((0, NP - N), (0, 0)))

    def pad_edges(x):
        return jnp.pad(x, ((0, E2 - E), (0, 0)))

    u2d = jnp.concatenate(
        [edge_index[0].astype(jnp.int32),
         jnp.full((E2 - E,), N, jnp.int32)]).reshape(E2 // CHUNK, CHUNK)
    v2d = jnp.concatenate(
        [edge_index[1].astype(jnp.int32),
         jnp.full((E2 - E,), N, jnp.int32)]).reshape(E2 // CHUNK, CHUNK)
    zeros32 = jnp.zeros((NP, HME), jnp.float32)
    zeros16 = jnp.zeros((NP, 16), jnp.float32)
    mass_pad = jnp.pad(massive, ((0, NP - N), (0, 0)), constant_values=1.0)

    wp16 = pad_cols(W_p, 16)
    wq16 = pad_cols(W_q, 16)
    wd16 = [pad_rows(W_msg[i][2 * HV + HE:], 16) for i in range(2)]
    wmu = [W_msg[i][:HV] for i in range(2)]
    wmv = [W_msg[i][HV:2 * HV] for i in range(2)]
    whu = [W_ham[i][:HV] for i in range(2)]
    whv = [W_ham[i][HV:2 * HV] for i in range(2)]

    hv, p16, q16, tu, tv = _init_nodes(
        atom_ftr, W_init_v, b_init_v[None], wp16, wq16,
        wmu[0], wmv[0], wd16[0], whu[0], whv[0])
    p16 = pad_nodes(p16)
    q16 = pad_nodes(q16)
    he, mhe, ce = _init_edges(
        bond_ftr, W_init_e, b_init_e[None],
        W_msg[0][2 * HV:2 * HV + HE], W_ham[0][2 * HV:])

    nxt = [1, 1]
    xs = dict(
        wuv1=jnp.stack([W_upd_v[i][:HV] for i in range(2)]),
        wuv2=jnp.stack([W_upd_v[i][HV:] for i in range(2)]),
        wa=jnp.stack([W_upd_e[i][:HE] for i in range(2)]),
        wb=jnp.stack([W_upd_e[i][HE:] for i in range(2)]),
        wc=jnp.stack([W_msg[j][2 * HV:2 * HV + HE] for j in nxt]),
        wdh=jnp.stack([W_ham[j][2 * HV:] for j in nxt]),
        wmu_n=jnp.stack([wmu[j] for j in nxt]),
        wmv_n=jnp.stack([wmv[j] for j in nxt]),
        wd16_n=jnp.stack([wd16[j] for j in nxt]),
        whu_n=jnp.stack([whu[j] for j in nxt]),
        whv_n=jnp.stack([whv[j] for j in nxt]),
    )

    def layer_body(carry, ws):
        hv, q16, p16, tu, tv, he, mhe, ce = carry
        ma, mb, coef, agga, aggb = _sc_edge_msg(
            pad_nodes(tu), pad_nodes(tv), pad_edges(mhe), pad_edges(ce),
            u2d, v2d, zeros32)
        t_hv = _upd_v(hv, agga[:, :N], aggb[:, :N], ws['wuv1'],
                      ws['wuv2'][:HME], ws['wuv2'][HME:])
        t_he, mhe_n, ce_n = _upd_e_proj(he, ma[:E], mb[:E], ws['wa'],
                                        ws['wb'][:HME], ws['wb'][HME:],
                                        ws['wc'], ws['wdh'])
        qq = _ham_pre(q16, p16, mass_pad)
        fa = _sc_ham_f(qq, coef, u2d, v2d, zeros32)
        qq2, p2 = _ham_mid(qq, p16, fa, mass_pad)
        fb = _sc_ham_f(qq2, coef, u2d, v2d, zeros32)
        q16, p16 = _ham_fin(qq2, p2, fb, mass_pad)
        tu_n, tv_n = _node_tables(t_hv, q16[:N], ws['wmu_n'], ws['wmv_n'],
                                  ws['wd16_n'], ws['whu_n'], ws['whv_n'])
        return (t_hv, q16, p16, tu_n, tv_n, t_he, mhe_n, ce_n), None

    carry, _ = lax.scan(layer_body, (hv, q16, p16, tu, tv, he, mhe, ce), xs,
                        length=2)
    hv, q16, p16 = carry[0], carry[1], carry[2]

    fp, conf8 = _readout(
        hv, p16[:N], q16[:N], mol_ids[:, None].astype(jnp.int32),
        W_fp, W_att[:HV], pad_rows(W_att[HV:HV + PQ], 16),
        pad_rows(W_att[HV + PQ:], 16), pad_rows(W_c1, 16), b_c1[None],
        pad_cols(W_c2, 8), pad_cols(b_c2[None], 8))
    return (fp, conf8[:, :PQ])
